# speedup vs baseline: 3.2938x; 3.2938x over previous
import jax
import jax.numpy as jnp
from jax.experimental import pallas as pl

_K = 64
_ROW = 32768
_BLOCK_ROWS = 8


def _topk_mask_kernel(x_ref, o_ref):
    x = x_ref[...]
    rows = x.shape[0]
    bits = jax.lax.bitcast_convert_type(x, jnp.int32)
    key = jnp.where(bits < 0, bits ^ jnp.int32(0x7FFFFFFF), bits)
    sign = jnp.int32(-2147483648)

    def value_bit(b, res_u):
        bit = jnp.left_shift(jnp.int32(1), jnp.int32(31) - b)
        cand_u = res_u | bit
        thr = cand_u ^ sign
        cnt = jnp.sum((key >= thr).astype(jnp.int32), axis=1, keepdims=True)
        return jnp.where(cnt >= _K, cand_u, res_u)

    res_u = jnp.zeros((rows, 1), jnp.int32)
    res_u = jax.lax.fori_loop(0, 32, value_bit, res_u, unroll=True)
    t = res_u ^ sign

    gt = key > t
    cnt_gt = jnp.sum(gt.astype(jnp.int32), axis=1, keepdims=True)
    need = _K - cnt_gt
    eq = key == t
    col = jax.lax.broadcasted_iota(jnp.int32, x.shape, 1)

    def idx_bit(b, carry):
        lo, hi = carry
        mid = jax.lax.div(lo + hi, jnp.int32(2))
        c = jnp.sum((eq & (col <= mid)).astype(jnp.int32), axis=1, keepdims=True)
        ok = c >= need
        return jnp.where(ok, lo, mid + 1), jnp.where(ok, mid, hi)

    lo = jnp.zeros((rows, 1), jnp.int32)
    hi = jnp.full((rows, 1), _ROW - 1, jnp.int32)
    lo, hi = jax.lax.fori_loop(0, 15, idx_bit, (lo, hi), unroll=True)

    mask = gt | (eq & (col <= lo))
    o_ref[...] = jnp.where(mask & (x > 0), x, 0.0)


@jax.jit
def kernel(x):
    grid = (x.shape[0] // _BLOCK_ROWS,)
    return pl.pallas_call(
        _topk_mask_kernel,
        grid=grid,
        in_specs=[pl.BlockSpec((_BLOCK_ROWS, _ROW), lambda r: (r, 0))],
        out_specs=pl.BlockSpec((_BLOCK_ROWS, _ROW), lambda r: (r, 0)),
        out_shape=jax.ShapeDtypeStruct(x.shape, x.dtype),
    )(x)


# speedup vs baseline: 6.6099x; 2.0068x over previous
import jax
import jax.numpy as jnp
from jax.experimental import pallas as pl

_K = 64
_ROW = 32768
_BLOCK_ROWS = 8


def _topk_mask_kernel(x_ref, o_ref):
    x = x_ref[...]
    rows = x.shape[0]
    bits = jax.lax.bitcast_convert_type(x, jnp.int32)
    key = jnp.where(bits < 0, bits ^ jnp.int32(0x7FFFFFFF), bits)
    sign = jnp.int32(-2147483648)

    def bit_cond(carry):
        b, _, cnt = carry
        return jnp.logical_and(b < 32, jnp.logical_not(jnp.all(cnt == _K)))

    def bit_body(carry):
        b, res_u, cnt = carry
        bit = jnp.left_shift(jnp.int32(1), jnp.int32(31) - b)
        cand_u = res_u | bit
        thr = cand_u ^ sign
        c = jnp.sum((key >= thr).astype(jnp.int32), axis=1, keepdims=True)
        take = c >= _K
        return (b + 1,
                jnp.where(take, cand_u, res_u),
                jnp.where(take, c, cnt))

    carry = (jnp.int32(0),
             jnp.zeros((rows, 1), jnp.int32),
             jnp.full((rows, 1), _ROW, jnp.int32))
    _, res_u, cnt = jax.lax.while_loop(bit_cond, bit_body, carry)
    t = res_u ^ sign

    exact = jnp.all(cnt == _K)

    @pl.when(exact)
    def _():
        o_ref[...] = jnp.where((key >= t) & (x > 0), x, 0.0)

    @pl.when(jnp.logical_not(exact))
    def _():
        gt = key > t
        cnt_gt = jnp.sum(gt.astype(jnp.int32), axis=1, keepdims=True)
        need = _K - cnt_gt
        eq = key == t
        col = jax.lax.broadcasted_iota(jnp.int32, x.shape, 1)

        def idx_bit(b, io):
            lo, hi = io
            mid = jax.lax.div(lo + hi, jnp.int32(2))
            c = jnp.sum((eq & (col <= mid)).astype(jnp.int32), axis=1,
                        keepdims=True)
            ok = c >= need
            return jnp.where(ok, lo, mid + 1), jnp.where(ok, mid, hi)

        lo = jnp.zeros((rows, 1), jnp.int32)
        hi = jnp.full((rows, 1), _ROW - 1, jnp.int32)
        lo, _ = jax.lax.fori_loop(0, 15, idx_bit, (lo, hi), unroll=True)
        mask = gt | (eq & (col <= lo))
        o_ref[...] = jnp.where(mask & (x > 0), x, 0.0)


@jax.jit
def kernel(x):
    grid = (x.shape[0] // _BLOCK_ROWS,)
    return pl.pallas_call(
        _topk_mask_kernel,
        grid=grid,
        in_specs=[pl.BlockSpec((_BLOCK_ROWS, _ROW), lambda r: (r, 0))],
        out_specs=pl.BlockSpec((_BLOCK_ROWS, _ROW), lambda r: (r, 0)),
        out_shape=jax.ShapeDtypeStruct(x.shape, x.dtype),
    )(x)


# speedup vs baseline: 13.0785x; 1.9786x over previous
import jax
import jax.numpy as jnp
from jax.experimental import pallas as pl

_K = 64
_ROW = 32768
_BLOCK_ROWS = 16
_QBITS = 20
_SIGN = -2147483648


def _count_ge(key, thr):
    return jnp.sum((key >= thr).astype(jnp.int32), axis=1, keepdims=True)


def _topk_mask_kernel(x_ref, o_ref):
    x = x_ref[...]
    rows = x.shape[0]
    bits = jax.lax.bitcast_convert_type(x, jnp.int32)
    key = jnp.where(bits < 0, bits ^ jnp.int32(0x7FFFFFFF), bits)
    sign = jnp.int32(_SIGN)

    parts = []
    for a in range(8):
        vs = [key[:, 4096 * a + 128 * i:4096 * a + 128 * (i + 1)]
              for i in range(32)]
        while len(vs) > 1:
            vs = [jnp.maximum(vs[j], vs[j + 1]) for j in range(0, len(vs), 2)]
        parts.append(vs[0])
    km = jnp.concatenate(parts, axis=1)

    lo_r = jnp.zeros((rows, 1), jnp.int32)
    hi_r = jnp.zeros((rows, 1), jnp.int32)
    for b in range(_QBITS):
        bit = jnp.int32(1 << (31 - b)) if b > 0 else sign
        lo_c = lo_r | bit
        hi_c = hi_r | bit
        c_lo = jnp.sum((km >= (lo_c ^ sign)).astype(jnp.int32), axis=1,
                       keepdims=True)
        c_hi = jnp.sum((km >= (hi_c ^ sign)).astype(jnp.int32), axis=1,
                       keepdims=True)
        lo_r = jnp.where(c_lo >= _K, lo_c, lo_r)
        hi_r = jnp.where(c_hi >= 2, hi_c, hi_r)
    lo_u = lo_r
    hi_u = hi_r | jnp.int32((1 << (32 - _QBITS)) - 1)

    d = lo_u ^ hi_u
    m = d
    for s in (1, 2, 4, 8, 16):
        m = m | jax.lax.shift_right_logical(m, jnp.int32(s))
    prefix = lo_u & jnp.invert(m)
    mp1 = m + 1
    e = jax.lax.shift_right_logical(
        jax.lax.bitcast_convert_type(mp1.astype(jnp.float32), jnp.int32),
        jnp.int32(23)) - 127
    bitpos0 = jnp.where(mp1 <= 0, jnp.where(m < 0, 31, 30), e - 1)

    cnt = _count_ge(key, prefix ^ sign)

    def bit_cond(carry):
        bp, _, cn = carry
        return jnp.any(jnp.logical_and(bp >= 0, cn != _K))

    def bit_body(carry):
        bp, res_u, cn = carry
        active = jnp.logical_and(bp >= 0, cn != _K)
        cand = res_u | jnp.left_shift(jnp.int32(1), jnp.maximum(bp, 0))
        c = _count_ge(key, cand ^ sign)
        take = jnp.logical_and(active, c >= _K)
        return (bp - active.astype(jnp.int32),
                jnp.where(take, cand, res_u),
                jnp.where(take, c, cn))

    _, res_u, cnt = jax.lax.while_loop(bit_cond, bit_body,
                                       (bitpos0, prefix, cnt))
    t = res_u ^ sign

    exact = jnp.all(cnt == _K)

    @pl.when(exact)
    def _():
        o_ref[...] = jnp.where((key >= t) & (x > 0), x, 0.0)

    @pl.when(jnp.logical_not(exact))
    def _():
        gt = key > t
        cnt_gt = jnp.sum(gt.astype(jnp.int32), axis=1, keepdims=True)
        need = _K - cnt_gt
        eq = key == t
        col = jax.lax.broadcasted_iota(jnp.int32, x.shape, 1)

        lo = jnp.zeros((rows, 1), jnp.int32)
        hi = jnp.full((rows, 1), _ROW - 1, jnp.int32)
        for _b in range(15):
            mid = jax.lax.div(lo + hi, jnp.int32(2))
            c = jnp.sum((eq & (col <= mid)).astype(jnp.int32), axis=1,
                        keepdims=True)
            ok = c >= need
            lo = jnp.where(ok, lo, mid + 1)
            hi = jnp.where(ok, mid, hi)
        mask = gt | (eq & (col <= lo))
        o_ref[...] = jnp.where(mask & (x > 0), x, 0.0)


@jax.jit
def kernel(x):
    grid = (x.shape[0] // _BLOCK_ROWS,)
    return pl.pallas_call(
        _topk_mask_kernel,
        grid=grid,
        in_specs=[pl.BlockSpec((_BLOCK_ROWS, _ROW), lambda r: (r, 0))],
        out_specs=pl.BlockSpec((_BLOCK_ROWS, _ROW), lambda r: (r, 0)),
        out_shape=jax.ShapeDtypeStruct(x.shape, x.dtype),
    )(x)


# speedup vs baseline: 13.6631x; 1.0447x over previous
import jax
import jax.numpy as jnp
from jax.experimental import pallas as pl

_K = 64
_ROW = 32768
_BLOCK_ROWS = 16
_QBITS = 20
_SIGN = -2147483648


def _count_ge(key, thr):
    return jnp.sum((key >= thr).astype(jnp.int32), axis=1, keepdims=True)


def _topk_mask_kernel(x_ref, o_ref):
    x = x_ref[...]
    rows = x.shape[0]
    bits = jax.lax.bitcast_convert_type(x, jnp.int32)
    key = jnp.where(bits < 0, bits ^ jnp.int32(0x7FFFFFFF), bits)
    sign = jnp.int32(_SIGN)

    parts = []
    for a in range(8):
        vs = [key[:, 4096 * a + 128 * i:4096 * a + 128 * (i + 1)]
              for i in range(32)]
        while len(vs) > 1:
            vs = [jnp.maximum(vs[j], vs[j + 1]) for j in range(0, len(vs), 2)]
        parts.append(vs[0])
    km = jnp.concatenate(parts, axis=1)

    lo_r = jnp.zeros((rows, 1), jnp.int32)
    hi_r = jnp.zeros((rows, 1), jnp.int32)
    for b in range(_QBITS):
        bit = jnp.int32(1 << (31 - b)) if b > 0 else sign
        lo_c = lo_r | bit
        hi_c = hi_r | bit
        c_lo = jnp.sum((km >= (lo_c ^ sign)).astype(jnp.int32), axis=1,
                       keepdims=True)
        c_hi = jnp.sum((km >= (hi_c ^ sign)).astype(jnp.int32), axis=1,
                       keepdims=True)
        lo_r = jnp.where(c_lo >= _K, lo_c, lo_r)
        hi_r = jnp.where(c_hi >= 2, hi_c, hi_r)
    lo_u = lo_r
    hi_u = hi_r | jnp.int32((1 << (32 - _QBITS)) - 1)

    d = lo_u ^ hi_u
    m = d
    for s in (1, 2, 4, 8, 16):
        m = m | jax.lax.shift_right_logical(m, jnp.int32(s))
    prefix = lo_u & jnp.invert(m)
    mp1 = m + 1
    e = jax.lax.shift_right_logical(
        jax.lax.bitcast_convert_type(mp1.astype(jnp.float32), jnp.int32),
        jnp.int32(23)) - 127
    bitpos0 = jnp.where(mp1 <= 0, jnp.where(m < 0, 31, 30), e - 1)

    cnt = jnp.full((rows, 1), _ROW, jnp.int32)

    def bit_cond(carry):
        bp, _, cn = carry
        return jnp.any(jnp.logical_and(bp >= 0, cn != _K))

    def bit_body(carry):
        bp, res_u, cn = carry
        active = jnp.logical_and(bp >= 0, cn != _K)
        cand = res_u | jnp.left_shift(jnp.int32(1), jnp.maximum(bp, 0))
        c = _count_ge(key, cand ^ sign)
        take = jnp.logical_and(active, c >= _K)
        return (bp - active.astype(jnp.int32),
                jnp.where(take, cand, res_u),
                jnp.where(take, c, cn))

    _, res_u, cnt = jax.lax.while_loop(bit_cond, bit_body,
                                       (bitpos0, prefix, cnt))
    t = res_u ^ sign

    exact = jnp.all(cnt == _K)

    @pl.when(exact)
    def _():
        o_ref[...] = jnp.where((key >= t) & (x > 0), x, 0.0)

    @pl.when(jnp.logical_not(exact))
    def _():
        gt = key > t
        cnt_gt = jnp.sum(gt.astype(jnp.int32), axis=1, keepdims=True)
        need = _K - cnt_gt
        eq = key == t
        col = jax.lax.broadcasted_iota(jnp.int32, x.shape, 1)

        lo = jnp.zeros((rows, 1), jnp.int32)
        hi = jnp.full((rows, 1), _ROW - 1, jnp.int32)
        for _b in range(15):
            mid = jax.lax.div(lo + hi, jnp.int32(2))
            c = jnp.sum((eq & (col <= mid)).astype(jnp.int32), axis=1,
                        keepdims=True)
            ok = c >= need
            lo = jnp.where(ok, lo, mid + 1)
            hi = jnp.where(ok, mid, hi)
        mask = gt | (eq & (col <= lo))
        o_ref[...] = jnp.where(mask & (x > 0), x, 0.0)


@jax.jit
def kernel(x):
    grid = (x.shape[0] // _BLOCK_ROWS,)
    return pl.pallas_call(
        _topk_mask_kernel,
        grid=grid,
        in_specs=[pl.BlockSpec((_BLOCK_ROWS, _ROW), lambda r: (r, 0))],
        out_specs=pl.BlockSpec((_BLOCK_ROWS, _ROW), lambda r: (r, 0)),
        out_shape=jax.ShapeDtypeStruct(x.shape, x.dtype),
    )(x)


# speedup vs baseline: 16.2207x; 1.1872x over previous
import jax
import jax.numpy as jnp
from jax.experimental import pallas as pl

_K = 64
_ROW = 32768
_BLOCK_ROWS = 32
_QBITS = 20
_SIGN = -2147483648


def _to_f(ku, sign):
    k = ku ^ sign
    fbits = jnp.where(k < 0, k ^ jnp.int32(0x7FFFFFFF), k)
    return jax.lax.bitcast_convert_type(fbits, jnp.float32)


def _count_ge_f(x, thr_f):
    return jnp.sum((x >= thr_f).astype(jnp.int32), axis=1, keepdims=True)


def _topk_mask_kernel(x_ref, o_ref):
    x = x_ref[...]
    rows = x.shape[0]
    sign = jnp.int32(_SIGN)

    m1_parts = []
    m2_parts = []
    for a in range(8):
        vs = [(x[:, 4096 * a + 128 * i:4096 * a + 128 * (i + 1)], None)
              for i in range(32)]
        while len(vs) > 1:
            nxt = []
            for j in range(0, len(vs), 2):
                a1, a2 = vs[j]
                b1, b2 = vs[j + 1]
                hi = jnp.maximum(a1, b1)
                lo = jnp.minimum(a1, b1)
                if a2 is None:
                    nxt.append((hi, lo))
                else:
                    nxt.append((hi, jnp.maximum(lo, jnp.maximum(a2, b2))))
            vs = nxt
        m1_parts.append(vs[0][0])
        m2_parts.append(vs[0][1])
    km = jnp.concatenate(m1_parts, axis=1)
    km2 = jnp.concatenate(m2_parts, axis=1)

    lo_r = jnp.zeros((rows, 1), jnp.int32)
    for b in range(_QBITS):
        bit = jnp.int32(1 << (31 - b)) if b > 0 else sign
        cand = lo_r | bit
        c = jnp.sum((km >= _to_f(cand, sign)).astype(jnp.int32), axis=1,
                    keepdims=True)
        lo_r = jnp.where(c >= _K, cand, lo_r)
    lo_u = lo_r

    m2max = jnp.max(km2, axis=1, keepdims=True)
    m2b = jax.lax.bitcast_convert_type(m2max, jnp.int32)
    m2max_u = jnp.where(m2b < 0, m2b ^ jnp.int32(0x7FFFFFFF), m2b) ^ sign
    q64_hi = lo_u | jnp.int32((1 << (32 - _QBITS)) - 1)
    hi_u = jnp.maximum(q64_hi ^ sign, m2max_u ^ sign) ^ sign

    d = lo_u ^ hi_u
    m = d
    for s in (1, 2, 4, 8, 16):
        m = m | jax.lax.shift_right_logical(m, jnp.int32(s))
    prefix = lo_u & jnp.invert(m)
    mp1 = m + 1
    e = jax.lax.shift_right_logical(
        jax.lax.bitcast_convert_type(mp1.astype(jnp.float32), jnp.int32),
        jnp.int32(23)) - 127
    bitpos0 = jnp.where(mp1 <= 0, jnp.where(m < 0, 31, 30), e - 1)

    cnt = _count_ge_f(x, _to_f(prefix, sign))

    def bit_cond(carry):
        bp, _, cn = carry
        return jnp.any(jnp.logical_and(bp >= 0, cn != _K))

    def bit_body(carry):
        bp, res_u, cn = carry
        active = jnp.logical_and(bp >= 0, cn != _K)
        cand = res_u | jnp.left_shift(jnp.int32(1), jnp.maximum(bp, 0))
        c = _count_ge_f(x, _to_f(cand, sign))
        take = jnp.logical_and(active, c >= _K)
        return (bp - active.astype(jnp.int32),
                jnp.where(take, cand, res_u),
                jnp.where(take, c, cn))

    _, res_u, cnt = jax.lax.while_loop(bit_cond, bit_body,
                                       (bitpos0, prefix, cnt))
    t_f = _to_f(res_u, sign)

    exact = jnp.all(cnt == _K)

    @pl.when(exact)
    def _():
        o_ref[...] = jnp.where((x >= t_f) & (x > 0), x, 0.0)

    @pl.when(jnp.logical_not(exact))
    def _():
        gt = x > t_f
        cnt_gt = jnp.sum(gt.astype(jnp.int32), axis=1, keepdims=True)
        need = _K - cnt_gt
        eq = x == t_f
        col = jax.lax.broadcasted_iota(jnp.int32, x.shape, 1)

        lo = jnp.zeros((rows, 1), jnp.int32)
        hi = jnp.full((rows, 1), _ROW - 1, jnp.int32)
        for _b in range(15):
            mid = jax.lax.div(lo + hi, jnp.int32(2))
            c = jnp.sum((eq & (col <= mid)).astype(jnp.int32), axis=1,
                        keepdims=True)
            ok = c >= need
            lo = jnp.where(ok, lo, mid + 1)
            hi = jnp.where(ok, mid, hi)
        mask = gt | (eq & (col <= lo))
        o_ref[...] = jnp.where(mask & (x > 0), x, 0.0)


@jax.jit
def kernel(x):
    grid = (x.shape[0] // _BLOCK_ROWS,)
    return pl.pallas_call(
        _topk_mask_kernel,
        grid=grid,
        in_specs=[pl.BlockSpec((_BLOCK_ROWS, _ROW), lambda r: (r, 0))],
        out_specs=pl.BlockSpec((_BLOCK_ROWS, _ROW), lambda r: (r, 0)),
        out_shape=jax.ShapeDtypeStruct(x.shape, x.dtype),
    )(x)


# speedup vs baseline: 18.9727x; 1.1697x over previous
import jax
import jax.numpy as jnp
from jax.experimental import pallas as pl

_K = 64
_ROW = 32768
_BLOCK_ROWS = 32
_QBITS = 20
_SIGN = -2147483648


def _to_f(ku, sign):
    k = ku ^ sign
    fbits = jnp.where(k < 0, k ^ jnp.int32(0x7FFFFFFF), k)
    return jax.lax.bitcast_convert_type(fbits, jnp.float32)


def _count_ge_f(x, thr_f):
    n = x.shape[1] // 128
    accs = [None] * 8
    for i in range(n):
        s = (x[:, 128 * i:128 * (i + 1)] >= thr_f).astype(jnp.int32)
        j = i % 8
        accs[j] = s if accs[j] is None else accs[j] + s
    accs = [a for a in accs if a is not None]
    while len(accs) > 1:
        accs = [accs[j] + accs[j + 1] if j + 1 < len(accs) else accs[j]
                for j in range(0, len(accs), 2)]
    return jnp.sum(accs[0], axis=1, keepdims=True)


def _interp_select(count_fn, rows, glo, ghi, clo, chi, kk, stop_at_k):
    kkf = jnp.float32(kk)

    def live(glo, ghi, clo):
        w1 = ghi - glo != 1
        if stop_at_k:
            return jnp.logical_and(clo != kk, w1)
        return w1

    def cond(carry):
        it, glo, ghi, clo, chi = carry
        return jnp.any(live(glo, ghi, clo))

    def body(carry):
        it, glo, ghi, clo, chi = carry
        active = live(glo, ghi, clo)
        w = ghi - glo
        p_mid = (glo & ghi) + jax.lax.shift_right_logical(glo ^ ghi,
                                                          jnp.int32(1))
        frac = (clo.astype(jnp.float32) - kkf) / jnp.maximum(
            (clo - chi).astype(jnp.float32), jnp.float32(1.0))
        off = (w.astype(jnp.float32) * frac).astype(jnp.int32)
        p_int = glo + jnp.clip(off, jnp.int32(1), jnp.maximum(w - 1, 1))
        use_int = jnp.logical_and(w > 0, (it & 1) == 0)
        p = jnp.where(use_int, p_int, p_mid)
        c = count_fn(p)
        ge = c >= kk
        upd_lo = jnp.logical_and(active, ge)
        upd_hi = jnp.logical_and(active, jnp.logical_not(ge))
        return (it + 1,
                jnp.where(upd_lo, p, glo),
                jnp.where(upd_hi, p, ghi),
                jnp.where(upd_lo, c, clo),
                jnp.where(upd_hi, c, chi))

    it0 = jnp.int32(0)
    _, glo, ghi, clo, chi = jax.lax.while_loop(
        cond, body, (it0, glo, ghi, clo, chi))
    return glo, ghi, clo


def _topk_mask_kernel(x_ref, o_ref):
    x = x_ref[...]
    rows = x.shape[0]
    sign = jnp.int32(_SIGN)

    m1_parts = []
    m2_parts = []
    for a in range(8):
        vs = [(x[:, 4096 * a + 128 * i:4096 * a + 128 * (i + 1)], None)
              for i in range(32)]
        while len(vs) > 1:
            nxt = []
            for j in range(0, len(vs), 2):
                a1, a2 = vs[j]
                b1, b2 = vs[j + 1]
                hi = jnp.maximum(a1, b1)
                lo = jnp.minimum(a1, b1)
                if a2 is None:
                    nxt.append((hi, lo))
                else:
                    nxt.append((hi, jnp.maximum(lo, jnp.maximum(a2, b2))))
            vs = nxt
        m1_parts.append(vs[0][0])
        m2_parts.append(vs[0][1])
    km = jnp.concatenate(m1_parts, axis=1)
    km2 = jnp.concatenate(m2_parts, axis=1)

    zero = jnp.zeros((rows, 1), jnp.int32)
    lo_r = zero
    for b in range(_QBITS):
        bit = jnp.int32(1 << (31 - b)) if b > 0 else sign
        cand = lo_r | bit
        c = _count_ge_f(km, _to_f(cand, sign))
        lo_r = jnp.where(c >= _K, cand, lo_r)
    lo_u = lo_r

    m2max = jnp.max(km2, axis=1, keepdims=True)
    m2b = jax.lax.bitcast_convert_type(m2max, jnp.int32)
    m2max_u = jnp.where(m2b < 0, m2b ^ jnp.int32(0x7FFFFFFF), m2b) ^ sign
    q64_hi = lo_u | jnp.int32((1 << (32 - _QBITS)) - 1)
    hi_u = jnp.maximum(q64_hi ^ sign, m2max_u ^ sign) ^ sign

    clo0 = _count_ge_f(x, _to_f(lo_u, sign))
    res_u, _, cnt = _interp_select(
        lambda p: _count_ge_f(x, _to_f(p, sign)), rows,
        lo_u, hi_u + 1, clo0, zero, _K, True)
    t_f = _to_f(res_u, sign)

    exact = jnp.all(cnt == _K)

    t_pos = jnp.maximum(t_f, jnp.float32(1e-45))

    @pl.when(exact)
    def _():
        o_ref[...] = jnp.where(x >= t_pos, x, 0.0)

    @pl.when(jnp.logical_not(exact))
    def _():
        gt = x > t_f
        cnt_gt = jnp.sum(gt.astype(jnp.int32), axis=1, keepdims=True)
        need = _K - cnt_gt
        eq = x == t_f
        col = jax.lax.broadcasted_iota(jnp.int32, x.shape, 1)

        lo = jnp.zeros((rows, 1), jnp.int32)
        hi = jnp.full((rows, 1), _ROW - 1, jnp.int32)
        for _b in range(15):
            mid = jax.lax.div(lo + hi, jnp.int32(2))
            c = jnp.sum((eq & (col <= mid)).astype(jnp.int32), axis=1,
                        keepdims=True)
            ok = c >= need
            lo = jnp.where(ok, lo, mid + 1)
            hi = jnp.where(ok, mid, hi)
        mask = gt | (eq & (col <= lo))
        o_ref[...] = jnp.where(mask & (x > 0), x, 0.0)


@jax.jit
def kernel(x):
    grid = (x.shape[0] // _BLOCK_ROWS,)
    return pl.pallas_call(
        _topk_mask_kernel,
        grid=grid,
        in_specs=[pl.BlockSpec((_BLOCK_ROWS, _ROW), lambda r: (r, 0))],
        out_specs=pl.BlockSpec((_BLOCK_ROWS, _ROW), lambda r: (r, 0)),
        out_shape=jax.ShapeDtypeStruct(x.shape, x.dtype),
    )(x)
